# baseline (speedup 1.0000x reference)
import functools

import jax
import jax.numpy as jnp
from jax import lax
from jax.experimental import pallas as pl
from jax.experimental.pallas import tpu as pltpu
from jax.experimental.pallas import tpu_sc as plsc

_LANES = 16


def _coef_sc_kernel(n_band: int, e_pad: int):
    n_pad = n_band + 128
    assert e_pad % 128 == 0
    n_chunks = e_pad // 128

    mesh = plsc.VectorSubcoreMesh(core_axis_name="c", subcore_axis_name="s")

    @functools.partial(
        pl.kernel,
        mesh=mesh,
        out_type=jax.ShapeDtypeStruct((3 * n_band,), jnp.float32),
        scratch_types=[
            [pltpu.VMEM((128,), jnp.int32) for _ in range(2 * n_chunks)],
            pltpu.VMEM((128,), jnp.float32),
            pltpu.VMEM((n_pad,), jnp.float32),
            pltpu.VMEM((n_pad,), jnp.float32),
            pltpu.VMEM_SHARED((n_pad,), jnp.float32),
            pltpu.VMEM_SHARED((n_pad,), jnp.float32),
            pltpu.VMEM_SHARED((n_pad,), jnp.float32),
            pltpu.VMEM((n_pad,), jnp.float32),
            [pltpu.VMEM((128,), jnp.float32) for _ in range(n_chunks)],
            [pltpu.VMEM((128,), jnp.float32) for _ in range(n_chunks)],
            [pltpu.VMEM((128,), jnp.float32) for _ in range(n_chunks)],
            [pltpu.VMEM((128,), jnp.float32) for _ in range(n_chunks)],
            pltpu.VMEM((3 * n_band,), jnp.float32),
            pltpu.SemaphoreType.DMA,
        ],
    )
    def coef_kernel(edges_hbm, out_hbm, idx, ones_v, deg_v, zeros_v,
                    sh_d, sh_l, sh_r, dinv_v, dvs, dvd, vl, vr, stage, sem):
        cid = lax.axis_index("c")
        sid = lax.axis_index("s")
        src_idx = idx[:n_chunks]
        dst_idx = idx[n_chunks:]

        @pl.when((cid == 0) & (sid == 0))
        def _():
            loads = [
                pltpu.async_copy(edges_hbm.at[pl.ds(k * 128, 128)],
                                 idx[k], sem)
                for k in range(2 * n_chunks)
            ]
            ones = jnp.ones((_LANES,), jnp.float32)
            zeros = jnp.zeros((_LANES,), jnp.float32)
            for i in range(128 // _LANES):
                ones_v[pl.ds(i * _LANES, _LANES)] = ones
            for i in range(n_pad // _LANES):
                deg_v[pl.ds(i * _LANES, _LANES)] = ones
                zeros_v[pl.ds(i * _LANES, _LANES)] = zeros
            for c in loads:
                c.wait()
            inits = [pltpu.async_copy(deg_v, sh_d, sem),
                     pltpu.async_copy(zeros_v, sh_l, sem),
                     pltpu.async_copy(zeros_v, sh_r, sem)]
            for c in inits:
                c.wait()
            for k in range(n_chunks):
                pltpu.sync_copy(ones_v, sh_d.at[dst_idx[k]], add=True)
            pltpu.sync_copy(sh_d, deg_v)
            for i in range(n_pad // _LANES):
                dg = deg_v[pl.ds(i * _LANES, _LANES)]
                y = 1.0 / dg
                for _ in range(6):
                    y = y * (1.5 - 0.5 * dg * y * y)
                dinv_v[pl.ds(i * _LANES, _LANES)] = y
                if i < n_band // _LANES:
                    stage[pl.ds(i * _LANES, _LANES)] = y * y
            pltpu.sync_copy(dinv_v, sh_d)
            gathers = []
            for k in range(n_chunks):
                gathers.append(pltpu.async_copy(sh_d.at[src_idx[k]],
                                                dvs[k], sem))
                gathers.append(pltpu.async_copy(sh_d.at[dst_idx[k]],
                                                dvd[k], sem))
            for c in gathers:
                c.wait()
            for k in range(n_chunks):
                for i in range(128 // _LANES):
                    s = src_idx[k][pl.ds(i * _LANES, _LANES)]
                    d = dst_idx[k][pl.ds(i * _LANES, _LANES)]
                    nrm = (dvs[k][pl.ds(i * _LANES, _LANES)]
                           * dvd[k][pl.ds(i * _LANES, _LANES)])
                    zf = jnp.zeros((_LANES,), jnp.float32)
                    vl[k][pl.ds(i * _LANES, _LANES)] = jnp.where(
                        s == d - 1, nrm, zf)
                    vr[k][pl.ds(i * _LANES, _LANES)] = jnp.where(
                        s == d + 1, nrm, zf)
            for k in range(n_chunks):
                pltpu.sync_copy(vl[k], sh_l.at[dst_idx[k]], add=True)
                pltpu.sync_copy(vr[k], sh_r.at[dst_idx[k]], add=True)
            outs = [pltpu.async_copy(sh_l.at[pl.ds(0, n_band)],
                                     stage.at[pl.ds(n_band, n_band)], sem),
                    pltpu.async_copy(sh_r.at[pl.ds(0, n_band)],
                                     stage.at[pl.ds(2 * n_band, n_band)],
                                     sem)]
            for c in outs:
                c.wait()
            pltpu.sync_copy(stage, out_hbm)

    return coef_kernel


def _gcn_dense_body(x_ref, w1_ref, w2_ref, b1_ref, b2_ref, out_ref,
                    *, hb: int):
    cx, _, wx = x_ref.shape[1:]
    xb = x_ref[0].astype(jnp.bfloat16).reshape(cx, hb * wx)
    w1 = w1_ref[...].astype(jnp.bfloat16)
    w2 = w2_ref[...].astype(jnp.bfloat16)
    z1 = lax.dot_general(w1, xb, (((0,), (0,)), ((), ())),
                         preferred_element_type=jnp.float32)
    h = jnp.maximum(z1 + b1_ref[...], 0.0).astype(jnp.bfloat16)
    z2 = lax.dot_general(w2, h, (((0,), (0,)), ((), ())),
                         preferred_element_type=jnp.float32)
    out_ref[0] = (z2 + b2_ref[...]).reshape(cx, hb, wx)


def _gcn_band_body(prev_ref, x_ref, w1_ref, w2_ref, b1_ref, b2_ref, coef_ref,
                   out_ref):
    a = coef_ref[0, :][None, :]
    lo = coef_ref[1, :][None, :]
    up = coef_ref[2, :][None, :]

    def band_mix(z):
        return (a * z
                + lo * jnp.roll(z, 1, axis=1)
                + up * jnp.roll(z, -1, axis=1))

    xr = x_ref[0, :, 0, :].astype(jnp.bfloat16)
    w1 = w1_ref[...].astype(jnp.bfloat16)
    w2 = w2_ref[...].astype(jnp.bfloat16)
    z1 = lax.dot_general(w1, xr, (((0,), (0,)), ((), ())),
                         preferred_element_type=jnp.float32)
    z1 = band_mix(z1)
    h = jnp.maximum(z1 + b1_ref[...], 0.0).astype(jnp.bfloat16)
    z2 = lax.dot_general(w2, h, (((0,), (0,)), ((), ())),
                         preferred_element_type=jnp.float32)
    z2 = band_mix(z2) + b2_ref[...]
    out_ref[0] = jnp.concatenate([z2[:, None, :], prev_ref[0, :, 1:, :]],
                                 axis=1)


def _gcn_tc_call(x, W1, b1, W2, b2, coef, hb: int, interpret: bool = False):
    Bx, Cx, Hx, Wx = x.shape
    hidden = W1.shape[1]
    n_band = coef.shape[1]
    assert n_band == Wx
    b1c = b1.reshape(hidden, 1)
    b2c = b2.reshape(Cx, 1)
    out_sds = jax.ShapeDtypeStruct((Bx, Cx, Hx, Wx), jnp.float32)
    wspecs = [
        pl.BlockSpec((Cx, hidden), lambda b, j: (0, 0)),
        pl.BlockSpec((hidden, Cx), lambda b, j: (0, 0)),
        pl.BlockSpec((hidden, 1), lambda b, j: (0, 0)),
        pl.BlockSpec((Cx, 1), lambda b, j: (0, 0)),
    ]
    dense = pl.pallas_call(
        functools.partial(_gcn_dense_body, hb=hb),
        grid=(Bx, Hx // hb),
        in_specs=[pl.BlockSpec((1, Cx, hb, Wx), lambda b, j: (b, 0, j, 0))]
        + wspecs,
        out_specs=pl.BlockSpec((1, Cx, hb, Wx), lambda b, j: (b, 0, j, 0)),
        out_shape=out_sds,
        interpret=interpret,
    )(x, W1, W2, b1c, b2c)
    row_spec = pl.BlockSpec((1, Cx, 8, Wx), lambda i: (0, 0, 0, 0))
    return pl.pallas_call(
        _gcn_band_body,
        grid=(1,),
        in_specs=[
            row_spec,
            row_spec,
            pl.BlockSpec((Cx, hidden), lambda i: (0, 0)),
            pl.BlockSpec((hidden, Cx), lambda i: (0, 0)),
            pl.BlockSpec((hidden, 1), lambda i: (0, 0)),
            pl.BlockSpec((Cx, 1), lambda i: (0, 0)),
            pl.BlockSpec((3, n_band), lambda i: (0, 0)),
        ],
        out_specs=row_spec,
        out_shape=out_sds,
        input_output_aliases={0: 0},
        interpret=interpret,
    )(dense, x, W1, W2, b1c, b2c, coef)


def kernel(x, W1, b1, W2, b2, edge_index):
    Bx, Cx, Hx, Wx = x.shape
    n_band = Cx
    e = edge_index.shape[1]
    e_pad = ((e + 127) // 128) * 128
    pad = jnp.full((e_pad - e,), n_band, jnp.int32)
    ei = edge_index.astype(jnp.int32)
    edges = jnp.concatenate([ei[0], pad, ei[1], pad])
    coef = _coef_sc_kernel(n_band, e_pad)(edges).reshape(3, n_band)

    return _gcn_tc_call(x, W1, b1, W2, b2, coef, hb=32)


# speedup vs baseline: 1.0244x; 1.0244x over previous
import functools

import jax
import jax.numpy as jnp
from jax import lax
from jax.experimental import pallas as pl
from jax.experimental.pallas import tpu as pltpu
from jax.experimental.pallas import tpu_sc as plsc

_LANES = 16


def _coef_sc_kernel(n_band: int, e_pad: int):
    n_pad = n_band + 128
    assert e_pad % 128 == 0
    n_chunks = e_pad // 128

    mesh = plsc.VectorSubcoreMesh(core_axis_name="c", subcore_axis_name="s")

    @functools.partial(
        pl.kernel,
        mesh=mesh,
        out_type=jax.ShapeDtypeStruct((3 * n_band,), jnp.float32),
        scratch_types=[
            [pltpu.VMEM((128,), jnp.int32) for _ in range(2 * n_chunks)],
            pltpu.VMEM((128,), jnp.float32),
            pltpu.VMEM((n_pad,), jnp.float32),
            pltpu.VMEM((n_pad,), jnp.float32),
            pltpu.VMEM_SHARED((n_pad,), jnp.float32),
            pltpu.VMEM_SHARED((n_pad,), jnp.float32),
            pltpu.VMEM_SHARED((n_pad,), jnp.float32),
            pltpu.VMEM((n_pad,), jnp.float32),
            [pltpu.VMEM((128,), jnp.float32) for _ in range(n_chunks)],
            [pltpu.VMEM((128,), jnp.float32) for _ in range(n_chunks)],
            [pltpu.VMEM((128,), jnp.float32) for _ in range(n_chunks)],
            [pltpu.VMEM((128,), jnp.float32) for _ in range(n_chunks)],
            pltpu.VMEM((3 * n_band,), jnp.float32),
            pltpu.SemaphoreType.DMA,
        ],
    )
    def coef_kernel(edges_hbm, out_hbm, idx, ones_v, deg_v, zeros_v,
                    sh_d, sh_l, sh_r, dinv_v, dvs, dvd, vl, vr, stage, sem):
        cid = lax.axis_index("c")
        sid = lax.axis_index("s")
        src_idx = idx[:n_chunks]
        dst_idx = idx[n_chunks:]

        @pl.when((cid == 0) & (sid == 0))
        def _():
            loads = [
                pltpu.async_copy(edges_hbm.at[pl.ds(k * 128, 128)],
                                 idx[k], sem)
                for k in range(2 * n_chunks)
            ]
            ones = jnp.ones((_LANES,), jnp.float32)
            zeros = jnp.zeros((_LANES,), jnp.float32)
            for i in range(128 // _LANES):
                ones_v[pl.ds(i * _LANES, _LANES)] = ones
            for i in range(n_pad // _LANES):
                deg_v[pl.ds(i * _LANES, _LANES)] = ones
                zeros_v[pl.ds(i * _LANES, _LANES)] = zeros
            for c in loads:
                c.wait()
            inits = [pltpu.async_copy(deg_v, sh_d, sem),
                     pltpu.async_copy(zeros_v, sh_l, sem),
                     pltpu.async_copy(zeros_v, sh_r, sem)]
            for c in inits:
                c.wait()
            for k in range(n_chunks):
                pltpu.sync_copy(ones_v, sh_d.at[dst_idx[k]], add=True)
            pltpu.sync_copy(sh_d, deg_v)
            for i in range(n_pad // _LANES):
                dg = deg_v[pl.ds(i * _LANES, _LANES)]
                y = 1.0 / dg
                for _ in range(6):
                    y = y * (1.5 - 0.5 * dg * y * y)
                dinv_v[pl.ds(i * _LANES, _LANES)] = y
                if i < n_band // _LANES:
                    stage[pl.ds(i * _LANES, _LANES)] = y * y
            pltpu.sync_copy(dinv_v, sh_d)
            gathers = []
            for k in range(n_chunks):
                gathers.append(pltpu.async_copy(sh_d.at[src_idx[k]],
                                                dvs[k], sem))
                gathers.append(pltpu.async_copy(sh_d.at[dst_idx[k]],
                                                dvd[k], sem))
            for c in gathers:
                c.wait()
            for k in range(n_chunks):
                for i in range(128 // _LANES):
                    s = src_idx[k][pl.ds(i * _LANES, _LANES)]
                    d = dst_idx[k][pl.ds(i * _LANES, _LANES)]
                    nrm = (dvs[k][pl.ds(i * _LANES, _LANES)]
                           * dvd[k][pl.ds(i * _LANES, _LANES)])
                    zf = jnp.zeros((_LANES,), jnp.float32)
                    vl[k][pl.ds(i * _LANES, _LANES)] = jnp.where(
                        s == d - 1, nrm, zf)
                    vr[k][pl.ds(i * _LANES, _LANES)] = jnp.where(
                        s == d + 1, nrm, zf)
            for k in range(n_chunks):
                pltpu.sync_copy(vl[k], sh_l.at[dst_idx[k]], add=True)
                pltpu.sync_copy(vr[k], sh_r.at[dst_idx[k]], add=True)
            outs = [pltpu.async_copy(sh_l.at[pl.ds(0, n_band)],
                                     stage.at[pl.ds(n_band, n_band)], sem),
                    pltpu.async_copy(sh_r.at[pl.ds(0, n_band)],
                                     stage.at[pl.ds(2 * n_band, n_band)],
                                     sem)]
            for c in outs:
                c.wait()
            pltpu.sync_copy(stage, out_hbm)

    return coef_kernel


def _gcn_dense_body(x_ref, w1_ref, w2_ref, b1_ref, b2_ref, out_ref,
                    *, hb: int):
    cx, _, wx = x_ref.shape[1:]
    xb = x_ref[0].astype(jnp.bfloat16).reshape(cx, hb * wx)
    w1 = w1_ref[...].astype(jnp.bfloat16)
    w2 = w2_ref[...].astype(jnp.bfloat16)
    z1 = lax.dot_general(w1, xb, (((0,), (0,)), ((), ())),
                         preferred_element_type=jnp.float32)
    h = jnp.maximum(z1 + b1_ref[...], 0.0).astype(jnp.bfloat16)
    z2 = lax.dot_general(w2, h, (((0,), (0,)), ((), ())),
                         preferred_element_type=jnp.float32)
    out_ref[0] = (z2 + b2_ref[...]).reshape(cx, hb, wx)


def _gcn_band_body(prev_ref, x_ref, w1_ref, w2_ref, b1_ref, b2_ref, coef_ref,
                   out_ref):
    a = coef_ref[0, :][None, :]
    lo = coef_ref[1, :][None, :]
    up = coef_ref[2, :][None, :]

    def band_mix(z):
        return (a * z
                + lo * jnp.roll(z, 1, axis=1)
                + up * jnp.roll(z, -1, axis=1))

    xr = x_ref[0, :, 0, :].astype(jnp.bfloat16)
    w1 = w1_ref[...].astype(jnp.bfloat16)
    w2 = w2_ref[...].astype(jnp.bfloat16)
    z1 = lax.dot_general(w1, xr, (((0,), (0,)), ((), ())),
                         preferred_element_type=jnp.float32)
    z1 = band_mix(z1)
    h = jnp.maximum(z1 + b1_ref[...], 0.0).astype(jnp.bfloat16)
    z2 = lax.dot_general(w2, h, (((0,), (0,)), ((), ())),
                         preferred_element_type=jnp.float32)
    z2 = band_mix(z2) + b2_ref[...]
    out_ref[0] = jnp.concatenate([z2[:, None, :], prev_ref[0, :, 1:, :]],
                                 axis=1)


def _gcn_tc_call(x, W1, b1, W2, b2, coef, hb: int, interpret: bool = False):
    Bx, Cx, Hx, Wx = x.shape
    hidden = W1.shape[1]
    n_band = coef.shape[1]
    assert n_band == Wx
    b1c = b1.reshape(hidden, 1)
    b2c = b2.reshape(Cx, 1)
    out_sds = jax.ShapeDtypeStruct((Bx, Cx, Hx, Wx), jnp.float32)
    wspecs = [
        pl.BlockSpec((Cx, hidden), lambda b, j: (0, 0)),
        pl.BlockSpec((hidden, Cx), lambda b, j: (0, 0)),
        pl.BlockSpec((hidden, 1), lambda b, j: (0, 0)),
        pl.BlockSpec((Cx, 1), lambda b, j: (0, 0)),
    ]
    dense = pl.pallas_call(
        functools.partial(_gcn_dense_body, hb=hb),
        grid=(Bx, Hx // hb),
        in_specs=[pl.BlockSpec((1, Cx, hb, Wx), lambda b, j: (b, 0, j, 0))]
        + wspecs,
        out_specs=pl.BlockSpec((1, Cx, hb, Wx), lambda b, j: (b, 0, j, 0)),
        out_shape=out_sds,
        interpret=interpret,
    )(x, W1, W2, b1c, b2c)
    row_spec = pl.BlockSpec((1, Cx, 8, Wx), lambda i: (0, 0, 0, 0))
    return pl.pallas_call(
        _gcn_band_body,
        grid=(1,),
        in_specs=[
            row_spec,
            row_spec,
            pl.BlockSpec((Cx, hidden), lambda i: (0, 0)),
            pl.BlockSpec((hidden, Cx), lambda i: (0, 0)),
            pl.BlockSpec((hidden, 1), lambda i: (0, 0)),
            pl.BlockSpec((Cx, 1), lambda i: (0, 0)),
            pl.BlockSpec((3, n_band), lambda i: (0, 0)),
        ],
        out_specs=row_spec,
        out_shape=out_sds,
        input_output_aliases={0: 0},
        interpret=interpret,
    )(dense, x, W1, W2, b1c, b2c, coef)


def kernel(x, W1, b1, W2, b2, edge_index):
    Bx, Cx, Hx, Wx = x.shape
    n_band = Cx
    e = edge_index.shape[1]
    e_pad = ((e + 127) // 128) * 128
    pad = jnp.full((e_pad - e,), n_band, jnp.int32)
    ei = edge_index.astype(jnp.int32)
    edges = jnp.concatenate([ei[0], pad, ei[1], pad])
    coef = _coef_sc_kernel(n_band, e_pad)(edges).reshape(3, n_band)

    return _gcn_tc_call(x, W1, b1, W2, b2, coef, hb=64)
